# baseline (speedup 1.0000x reference)
import functools

import jax
import jax.numpy as jnp
from jax import lax
from jax.experimental import pallas as pl
from jax.experimental.pallas import tpu as pltpu
from jax.experimental.pallas import tpu_sc as plsc

DIM = 2048
INTER = 1408
N_EXP = 16
TOP_K = 2
S = 2048
M = 128
NB = 48
CAP = M * NB
TB = 256


def _gate_body(x_ref, gw_ref, wts_ref, pos_ref, be_ref):
    xb = x_ref[...]
    gw = gw_ref[...]
    logits = lax.dot_general(xb, gw, (((1,), (1,)), ((), ())),
                             preferred_element_type=jnp.float32)
    m = jnp.max(logits, axis=-1, keepdims=True)
    ex = jnp.exp(logits - m)
    sc = ex / jnp.sum(ex, axis=-1, keepdims=True)

    lane = lax.broadcasted_iota(jnp.int32, (S, N_EXP), 1)
    v1 = jnp.max(sc, axis=-1, keepdims=True)
    i1 = jnp.min(jnp.where(sc == v1, lane, N_EXP), axis=-1, keepdims=True)
    sc2 = jnp.where(lane == i1, -1.0, sc)
    v2 = jnp.max(sc2, axis=-1, keepdims=True)
    i2 = jnp.min(jnp.where(sc2 == v2, lane, N_EXP), axis=-1, keepdims=True)
    wts_ref[...] = jnp.concatenate([v1, v2], axis=1)

    oh1 = lane == i1
    oh2 = lane == i2
    ohf = (oh1 | oh2).astype(jnp.float32)

    r = lax.broadcasted_iota(jnp.int32, (S, S), 0)
    c = lax.broadcasted_iota(jnp.int32, (S, S), 1)
    tri = (c < r).astype(jnp.float32)
    excl = lax.dot_general(tri, ohf, (((1,), (0,)), ((), ())),
                           preferred_element_type=jnp.float32)

    counts = jnp.sum(ohf, axis=0, keepdims=True)
    cp = jnp.floor((counts + (M - 1)) * (1.0 / M)) * M
    er = lax.broadcasted_iota(jnp.int32, (N_EXP, N_EXP), 0)
    ec = lax.broadcasted_iota(jnp.int32, (N_EXP, N_EXP), 1)
    ut = (er < ec).astype(jnp.float32)
    start = lax.dot_general(cp, ut, (((1,), (0,)), ((), ())),
                            preferred_element_type=jnp.float32)

    dest = start + excl
    p1 = jnp.sum(jnp.where(oh1, dest, 0.0), axis=-1, keepdims=True)
    p2 = jnp.sum(jnp.where(oh2, dest, 0.0), axis=-1, keepdims=True)
    pos_ref[...] = jnp.concatenate([p1, p2], axis=1).astype(jnp.int32)

    rb = (lax.broadcasted_iota(jnp.int32, (1, NB), 1) * M).astype(jnp.float32)
    rbt = jnp.broadcast_to(rb.reshape(NB, 1), (NB, N_EXP))
    sb = jnp.broadcast_to(start, (NB, N_EXP))
    cb = jnp.broadcast_to(cp, (NB, N_EXP))
    act = (rbt >= sb) & (rbt < sb + cb)
    ev = lax.broadcasted_iota(jnp.int32, (NB, N_EXP), 1).astype(jnp.float32)
    be = jnp.sum(jnp.where(act, ev + 1.0, 0.0), axis=-1, keepdims=False) - 1.0
    be_ref[...] = be.astype(jnp.int32).reshape(1, NB)


def _gate_call(x2d, gate_w):
    return pl.pallas_call(
        _gate_body,
        out_shape=(
            jax.ShapeDtypeStruct((S, TOP_K), jnp.float32),
            jax.ShapeDtypeStruct((S, TOP_K), jnp.int32),
            jax.ShapeDtypeStruct((1, NB), jnp.int32),
        ),
    )(x2d, gate_w)


def _make_sc_scatter():
    info = plsc.get_sparse_core_info()
    nc, ns = info.num_cores, info.num_subcores
    nw = nc * ns
    tpw = S // nw
    chunks = tpw // 16
    mesh = plsc.VectorSubcoreMesh(core_axis_name="c", subcore_axis_name="s")

    @functools.partial(
        pl.kernel, mesh=mesh,
        out_type=jax.ShapeDtypeStruct((CAP, DIM), jnp.float32),
        scratch_types=(
            [pltpu.VMEM((16, DIM), jnp.float32)] * 2
            + [pltpu.VMEM((16,), jnp.int32)] * (2 * chunks)
            + [pltpu.SemaphoreType.DMA] * 3
        ),
    )
    def scatter_kernel(x_hbm, p0_hbm, p1_hbm, xs_hbm, *scr):
        xa, xb = scr[0], scr[1]
        idx = scr[2:2 + 2 * chunks]
        lsem, ssem, isem = scr[-3], scr[-2], scr[-1]
        wid = lax.axis_index("s") * nc + lax.axis_index("c")
        base = wid * tpw
        for cc in range(chunks):
            pltpu.async_copy(p0_hbm.at[pl.ds(base + cc * 16, 16)],
                             idx[2 * cc], isem)
            pltpu.async_copy(p1_hbm.at[pl.ds(base + cc * 16, 16)],
                             idx[2 * cc + 1], isem)
        bufs = [xa, xb]
        lds = [None, None]
        lds[0] = pltpu.async_copy(x_hbm.at[pl.ds(base, 16)], xa, lsem)
        for cc in range(chunks):
            pltpu.make_async_copy(p0_hbm.at[pl.ds(base + cc * 16, 16)],
                                  idx[2 * cc], isem).wait()
            pltpu.make_async_copy(p1_hbm.at[pl.ds(base + cc * 16, 16)],
                                  idx[2 * cc + 1], isem).wait()
        sc_on_buf = [[], []]
        for cc in range(chunks):
            b = cc % 2
            nb_ = (cc + 1) % 2
            for s in sc_on_buf[nb_]:
                s.wait()
            sc_on_buf[nb_] = []
            if cc + 1 < chunks:
                lds[nb_] = pltpu.async_copy(
                    x_hbm.at[pl.ds(base + (cc + 1) * 16, 16)], bufs[nb_],
                    lsem)
            lds[b].wait()
            sc_on_buf[b] = [
                pltpu.async_copy(bufs[b], xs_hbm.at[idx[2 * cc]], ssem),
                pltpu.async_copy(bufs[b], xs_hbm.at[idx[2 * cc + 1]], ssem),
            ]
        for sl in sc_on_buf:
            for s in sl:
                s.wait()

    return scatter_kernel


def _ffn1_body(be_ref, xs_ref, w1_ref, w3_ref, h_ref):
    i = pl.program_id(0)

    @pl.when(be_ref[0, i] >= 0)
    def _():
        xb = xs_ref[...]
        h1 = lax.dot_general(xb, w1_ref[0], (((1,), (1,)), ((), ())),
                             preferred_element_type=jnp.float32)
        h3 = lax.dot_general(xb, w3_ref[0], (((1,), (1,)), ((), ())),
                             preferred_element_type=jnp.float32)
        h_ref[...] = ((h1 / (1.0 + jnp.exp(-h1))) * h3).astype(jnp.bfloat16)


def _ffn2_body(be_ref, h_ref, w2_ref, out_ref):
    i = pl.program_id(0)

    @pl.when(be_ref[0, i] >= 0)
    def _():
        hb = h_ref[...].astype(jnp.float32)
        out_ref[...] = lax.dot_general(
            hb, w2_ref[0], (((1,), (1,)), ((), ())),
            preferred_element_type=jnp.float32)


def _expert_spec():
    return lambda i, be: (jnp.maximum(be[0, i], 0), 0, 0)


def _ffn_call(be, xs, w1, w3, w2):
    gs1 = pltpu.PrefetchScalarGridSpec(
        num_scalar_prefetch=1,
        grid=(NB,),
        in_specs=[
            pl.BlockSpec((M, DIM), lambda i, be: (i, 0)),
            pl.BlockSpec((1, INTER, DIM), _expert_spec()),
            pl.BlockSpec((1, INTER, DIM), _expert_spec()),
        ],
        out_specs=pl.BlockSpec((M, INTER), lambda i, be: (i, 0)),
    )
    h = pl.pallas_call(
        _ffn1_body,
        grid_spec=gs1,
        out_shape=jax.ShapeDtypeStruct((CAP, INTER), jnp.bfloat16),
    )(be, xs, w1, w3)
    gs2 = pltpu.PrefetchScalarGridSpec(
        num_scalar_prefetch=1,
        grid=(NB,),
        in_specs=[
            pl.BlockSpec((M, INTER), lambda i, be: (i, 0)),
            pl.BlockSpec((1, DIM, INTER), _expert_spec()),
        ],
        out_specs=pl.BlockSpec((M, DIM), lambda i, be: (i, 0)),
    )
    return pl.pallas_call(
        _ffn2_body,
        grid_spec=gs2,
        out_shape=jax.ShapeDtypeStruct((CAP, DIM), jnp.float32),
    )(be, h, w2)


def _make_sc_gather():
    info = plsc.get_sparse_core_info()
    nc, ns = info.num_cores, info.num_subcores
    nw = nc * ns
    tpw = S // nw
    chunks = tpw // 16
    mesh = plsc.VectorSubcoreMesh(core_axis_name="c", subcore_axis_name="s")

    @functools.partial(
        pl.kernel, mesh=mesh,
        out_type=(
            jax.ShapeDtypeStruct((S, DIM), jnp.float32),
            jax.ShapeDtypeStruct((S, DIM), jnp.float32),
        ),
        scratch_types=(
            [pltpu.VMEM((16, DIM), jnp.float32)] * 2
            + [pltpu.VMEM((tpw,), jnp.int32)] * 2
            + [pltpu.SemaphoreType.DMA] * 3
        ),
    )
    def gather_kernel(ys_hbm, p0_hbm, p1_hbm, g0_hbm, g1_hbm, *scr):
        bufs = [scr[0], scr[1]]
        i0, i1 = scr[2], scr[3]
        gsem, ssem, isem = scr[-3], scr[-2], scr[-1]
        wid = lax.axis_index("s") * nc + lax.axis_index("c")
        base = wid * tpw
        pltpu.async_copy(p0_hbm.at[pl.ds(base, tpw)], i0, isem)
        pltpu.async_copy(p1_hbm.at[pl.ds(base, tpw)], i1, isem)
        pltpu.make_async_copy(p0_hbm.at[pl.ds(base, tpw)], i0, isem).wait()
        pltpu.make_async_copy(p1_hbm.at[pl.ds(base, tpw)], i1, isem).wait()
        stores = [[], []]
        for idx, g_hbm in ((i0, g0_hbm), (i1, g1_hbm)):
            prev = None
            for cc in range(chunks):
                b = cc % 2
                for st in stores[b]:
                    st.wait()
                stores[b] = []
                gc = pltpu.async_copy(
                    ys_hbm.at[idx.at[pl.ds(cc * 16, 16)]], bufs[b], gsem)
                if prev is not None:
                    pc, pg = prev
                    pg.wait()
                    stores[pc % 2] = [pltpu.async_copy(
                        bufs[pc % 2],
                        g_hbm.at[pl.ds(base + pc * 16, 16)], ssem)]
                prev = (cc, gc)
            pc, pg = prev
            pg.wait()
            stores[pc % 2] = [pltpu.async_copy(
                bufs[pc % 2], g_hbm.at[pl.ds(base + pc * 16, 16)], ssem)]
        for sl in stores:
            for st in sl:
                st.wait()

    return gather_kernel


def _shared1_body(x_ref, ws1_ref, ws3_ref, h_ref):
    xb = x_ref[...]
    h1 = lax.dot_general(xb, ws1_ref[...], (((1,), (1,)), ((), ())),
                         preferred_element_type=jnp.float32)
    h3 = lax.dot_general(xb, ws3_ref[...], (((1,), (1,)), ((), ())),
                         preferred_element_type=jnp.float32)
    h_ref[...] = (h1 / (1.0 + jnp.exp(-h1))) * h3


def _combine_body(h_ref, ws2_ref, g0_ref, g1_ref, wts_ref, out_ref):
    z = lax.dot_general(h_ref[...], ws2_ref[...], (((1,), (1,)), ((), ())),
                        preferred_element_type=jnp.float32)
    w = wts_ref[...]
    out_ref[...] = (z + w[:, 0:1] * g0_ref[...] + w[:, 1:2] * g1_ref[...])


def _combine_call(x2d, ws1, ws3, ws2, g0, g1, wts):
    nblk = S // TB
    hz = pl.pallas_call(
        _shared1_body,
        grid=(nblk,),
        in_specs=[
            pl.BlockSpec((TB, DIM), lambda i: (i, 0)),
            pl.BlockSpec((INTER, DIM), lambda i: (0, 0)),
            pl.BlockSpec((INTER, DIM), lambda i: (0, 0)),
        ],
        out_specs=pl.BlockSpec((TB, INTER), lambda i: (i, 0)),
        out_shape=jax.ShapeDtypeStruct((S, INTER), jnp.float32),
    )(x2d, ws1, ws3)
    return pl.pallas_call(
        _combine_body,
        grid=(nblk,),
        in_specs=[
            pl.BlockSpec((TB, INTER), lambda i: (i, 0)),
            pl.BlockSpec((DIM, INTER), lambda i: (0, 0)),
            pl.BlockSpec((TB, DIM), lambda i: (i, 0)),
            pl.BlockSpec((TB, DIM), lambda i: (i, 0)),
            pl.BlockSpec((TB, TOP_K), lambda i: (i, 0)),
        ],
        out_specs=pl.BlockSpec((TB, DIM), lambda i: (i, 0)),
        out_shape=jax.ShapeDtypeStruct((S, DIM), jnp.float32),
    )(hz, ws2, g0, g1, wts)


def kernel(x, gate_w, w1, w2, w3, ws1, ws2, ws3):
    shape = x.shape
    x2d = x.reshape(S, DIM)
    wts, pos, be = _gate_call(x2d, gate_w)
    pos0 = pos[:, 0]
    pos1 = pos[:, 1]
    xs = _make_sc_scatter()(x2d, pos0, pos1)
    ys = _ffn_call(be, xs, w1, w3, w2)
    g0, g1 = _make_sc_gather()(ys, pos0, pos1)
    y = _combine_call(x2d, ws1, ws3, ws2, g0, g1, wts)
    return y.reshape(shape)


# speedup vs baseline: 1.2673x; 1.2673x over previous
import functools

import jax
import jax.numpy as jnp
from jax import lax
from jax.experimental import pallas as pl
from jax.experimental.pallas import tpu as pltpu
from jax.experimental.pallas import tpu_sc as plsc

DIM = 2048
INTER = 1408
N_EXP = 16
TOP_K = 2
S = 2048
M = 512
NB = 24
CAP = M * NB
TB = 256


def _gate_body(x_ref, gw_ref, wts_ref, pos_ref, be_ref):
    xb = x_ref[...]
    gw = gw_ref[...]
    logits = lax.dot_general(xb, gw, (((1,), (1,)), ((), ())),
                             preferred_element_type=jnp.float32)
    m = jnp.max(logits, axis=-1, keepdims=True)
    ex = jnp.exp(logits - m)
    sc = ex / jnp.sum(ex, axis=-1, keepdims=True)

    lane = lax.broadcasted_iota(jnp.int32, (S, N_EXP), 1)
    v1 = jnp.max(sc, axis=-1, keepdims=True)
    i1 = jnp.min(jnp.where(sc == v1, lane, N_EXP), axis=-1, keepdims=True)
    sc2 = jnp.where(lane == i1, -1.0, sc)
    v2 = jnp.max(sc2, axis=-1, keepdims=True)
    i2 = jnp.min(jnp.where(sc2 == v2, lane, N_EXP), axis=-1, keepdims=True)
    wts_ref[...] = jnp.concatenate([v1, v2], axis=1)

    oh1 = lane == i1
    oh2 = lane == i2
    ohf = (oh1 | oh2).astype(jnp.float32)

    r = lax.broadcasted_iota(jnp.int32, (S, S), 0)
    c = lax.broadcasted_iota(jnp.int32, (S, S), 1)
    tri = (c < r).astype(jnp.float32)
    excl = lax.dot_general(tri, ohf, (((1,), (0,)), ((), ())),
                           preferred_element_type=jnp.float32)

    counts = jnp.sum(ohf, axis=0, keepdims=True)
    cp = jnp.floor((counts + (M - 1)) * (1.0 / M)) * M
    er = lax.broadcasted_iota(jnp.int32, (N_EXP, N_EXP), 0)
    ec = lax.broadcasted_iota(jnp.int32, (N_EXP, N_EXP), 1)
    ut = (er < ec).astype(jnp.float32)
    start = lax.dot_general(cp, ut, (((1,), (0,)), ((), ())),
                            preferred_element_type=jnp.float32)

    dest = start + excl
    p1 = jnp.sum(jnp.where(oh1, dest, 0.0), axis=-1, keepdims=True)
    p2 = jnp.sum(jnp.where(oh2, dest, 0.0), axis=-1, keepdims=True)
    pos_ref[...] = jnp.concatenate([p1, p2], axis=1).astype(jnp.int32)

    rb = (lax.broadcasted_iota(jnp.int32, (1, NB), 1) * M).astype(jnp.float32)
    rbt = jnp.broadcast_to(rb.reshape(NB, 1), (NB, N_EXP))
    sb = jnp.broadcast_to(start, (NB, N_EXP))
    cb = jnp.broadcast_to(cp, (NB, N_EXP))
    act = (rbt >= sb) & (rbt < sb + cb)
    ev = lax.broadcasted_iota(jnp.int32, (NB, N_EXP), 1).astype(jnp.float32)
    be = jnp.sum(jnp.where(act, ev + 1.0, 0.0), axis=-1, keepdims=False) - 1.0
    be_ref[...] = be.astype(jnp.int32).reshape(1, NB)


def _gate_call(x2d, gate_w):
    return pl.pallas_call(
        _gate_body,
        out_shape=(
            jax.ShapeDtypeStruct((S, TOP_K), jnp.float32),
            jax.ShapeDtypeStruct((S, TOP_K), jnp.int32),
            jax.ShapeDtypeStruct((1, NB), jnp.int32),
        ),
    )(x2d, gate_w)


def _make_sc_scatter():
    info = plsc.get_sparse_core_info()
    nc, ns = info.num_cores, info.num_subcores
    nw = nc * ns
    tpw = S // nw
    chunks = tpw // 16
    mesh = plsc.VectorSubcoreMesh(core_axis_name="c", subcore_axis_name="s")

    @functools.partial(
        pl.kernel, mesh=mesh,
        out_type=jax.ShapeDtypeStruct((CAP, DIM), jnp.float32),
        scratch_types=(
            [pltpu.VMEM((16, DIM), jnp.float32)] * 2
            + [pltpu.VMEM((16,), jnp.int32)] * (2 * chunks)
            + [pltpu.SemaphoreType.DMA] * 3
        ),
    )
    def scatter_kernel(x_hbm, p0_hbm, p1_hbm, xs_hbm, *scr):
        xa, xb = scr[0], scr[1]
        idx = scr[2:2 + 2 * chunks]
        lsem, ssem, isem = scr[-3], scr[-2], scr[-1]
        wid = lax.axis_index("s") * nc + lax.axis_index("c")
        base = wid * tpw
        for cc in range(chunks):
            pltpu.async_copy(p0_hbm.at[pl.ds(base + cc * 16, 16)],
                             idx[2 * cc], isem)
            pltpu.async_copy(p1_hbm.at[pl.ds(base + cc * 16, 16)],
                             idx[2 * cc + 1], isem)
        bufs = [xa, xb]
        lds = [None, None]
        lds[0] = pltpu.async_copy(x_hbm.at[pl.ds(base, 16)], xa, lsem)
        for cc in range(chunks):
            pltpu.make_async_copy(p0_hbm.at[pl.ds(base + cc * 16, 16)],
                                  idx[2 * cc], isem).wait()
            pltpu.make_async_copy(p1_hbm.at[pl.ds(base + cc * 16, 16)],
                                  idx[2 * cc + 1], isem).wait()
        sc_on_buf = [[], []]
        for cc in range(chunks):
            b = cc % 2
            nb_ = (cc + 1) % 2
            for s in sc_on_buf[nb_]:
                s.wait()
            sc_on_buf[nb_] = []
            if cc + 1 < chunks:
                lds[nb_] = pltpu.async_copy(
                    x_hbm.at[pl.ds(base + (cc + 1) * 16, 16)], bufs[nb_],
                    lsem)
            lds[b].wait()
            sc_on_buf[b] = [
                pltpu.async_copy(bufs[b], xs_hbm.at[idx[2 * cc]], ssem),
                pltpu.async_copy(bufs[b], xs_hbm.at[idx[2 * cc + 1]], ssem),
            ]
        for sl in sc_on_buf:
            for s in sl:
                s.wait()

    return scatter_kernel


def _ffn1_body(be_ref, xs_ref, w1_ref, w3_ref, h_ref):
    i = pl.program_id(0)

    @pl.when(be_ref[0, i] >= 0)
    def _():
        xb = xs_ref[...]
        h1 = lax.dot_general(xb, w1_ref[0], (((1,), (1,)), ((), ())),
                             preferred_element_type=jnp.float32)
        h3 = lax.dot_general(xb, w3_ref[0], (((1,), (1,)), ((), ())),
                             preferred_element_type=jnp.float32)
        h_ref[...] = ((h1 / (1.0 + jnp.exp(-h1))) * h3).astype(jnp.bfloat16)


def _ffn2_body(be_ref, h_ref, w2_ref, out_ref):
    i = pl.program_id(0)

    @pl.when(be_ref[0, i] >= 0)
    def _():
        hb = h_ref[...].astype(jnp.float32)
        out_ref[...] = lax.dot_general(
            hb, w2_ref[0], (((1,), (1,)), ((), ())),
            preferred_element_type=jnp.float32)


def _expert_spec():
    return lambda i, be: (jnp.maximum(be[0, i], 0), 0, 0)


def _ffn_call(be, xs, w1, w3, w2):
    gs1 = pltpu.PrefetchScalarGridSpec(
        num_scalar_prefetch=1,
        grid=(NB,),
        in_specs=[
            pl.BlockSpec((M, DIM), lambda i, be: (i, 0)),
            pl.BlockSpec((1, INTER, DIM), _expert_spec()),
            pl.BlockSpec((1, INTER, DIM), _expert_spec()),
        ],
        out_specs=pl.BlockSpec((M, INTER), lambda i, be: (i, 0)),
    )
    h = pl.pallas_call(
        _ffn1_body,
        grid_spec=gs1,
        out_shape=jax.ShapeDtypeStruct((CAP, INTER), jnp.bfloat16),
    )(be, xs, w1, w3)
    gs2 = pltpu.PrefetchScalarGridSpec(
        num_scalar_prefetch=1,
        grid=(NB,),
        in_specs=[
            pl.BlockSpec((M, INTER), lambda i, be: (i, 0)),
            pl.BlockSpec((1, DIM, INTER), _expert_spec()),
        ],
        out_specs=pl.BlockSpec((M, DIM), lambda i, be: (i, 0)),
    )
    return pl.pallas_call(
        _ffn2_body,
        grid_spec=gs2,
        out_shape=jax.ShapeDtypeStruct((CAP, DIM), jnp.float32),
    )(be, h, w2)


def _make_sc_gather():
    info = plsc.get_sparse_core_info()
    nc, ns = info.num_cores, info.num_subcores
    nw = nc * ns
    tpw = S // nw
    chunks = tpw // 16
    mesh = plsc.VectorSubcoreMesh(core_axis_name="c", subcore_axis_name="s")

    @functools.partial(
        pl.kernel, mesh=mesh,
        out_type=(
            jax.ShapeDtypeStruct((S, DIM), jnp.float32),
            jax.ShapeDtypeStruct((S, DIM), jnp.float32),
        ),
        scratch_types=(
            [pltpu.VMEM((16, DIM), jnp.float32)] * 2
            + [pltpu.VMEM((tpw,), jnp.int32)] * 2
            + [pltpu.SemaphoreType.DMA] * 3
        ),
    )
    def gather_kernel(ys_hbm, p0_hbm, p1_hbm, g0_hbm, g1_hbm, *scr):
        bufs = [scr[0], scr[1]]
        i0, i1 = scr[2], scr[3]
        gsem, ssem, isem = scr[-3], scr[-2], scr[-1]
        wid = lax.axis_index("s") * nc + lax.axis_index("c")
        base = wid * tpw
        pltpu.async_copy(p0_hbm.at[pl.ds(base, tpw)], i0, isem)
        pltpu.async_copy(p1_hbm.at[pl.ds(base, tpw)], i1, isem)
        pltpu.make_async_copy(p0_hbm.at[pl.ds(base, tpw)], i0, isem).wait()
        pltpu.make_async_copy(p1_hbm.at[pl.ds(base, tpw)], i1, isem).wait()
        stores = [[], []]
        for idx, g_hbm in ((i0, g0_hbm), (i1, g1_hbm)):
            prev = None
            for cc in range(chunks):
                b = cc % 2
                for st in stores[b]:
                    st.wait()
                stores[b] = []
                gc = pltpu.async_copy(
                    ys_hbm.at[idx.at[pl.ds(cc * 16, 16)]], bufs[b], gsem)
                if prev is not None:
                    pc, pg = prev
                    pg.wait()
                    stores[pc % 2] = [pltpu.async_copy(
                        bufs[pc % 2],
                        g_hbm.at[pl.ds(base + pc * 16, 16)], ssem)]
                prev = (cc, gc)
            pc, pg = prev
            pg.wait()
            stores[pc % 2] = [pltpu.async_copy(
                bufs[pc % 2], g_hbm.at[pl.ds(base + pc * 16, 16)], ssem)]
        for sl in stores:
            for st in sl:
                st.wait()

    return gather_kernel


def _shared1_body(x_ref, ws1_ref, ws3_ref, h_ref):
    xb = x_ref[...]
    h1 = lax.dot_general(xb, ws1_ref[...], (((1,), (1,)), ((), ())),
                         preferred_element_type=jnp.float32)
    h3 = lax.dot_general(xb, ws3_ref[...], (((1,), (1,)), ((), ())),
                         preferred_element_type=jnp.float32)
    h_ref[...] = (h1 / (1.0 + jnp.exp(-h1))) * h3


def _combine_body(h_ref, ws2_ref, g0_ref, g1_ref, wts_ref, out_ref):
    z = lax.dot_general(h_ref[...], ws2_ref[...], (((1,), (1,)), ((), ())),
                        preferred_element_type=jnp.float32)
    w = wts_ref[...]
    out_ref[...] = (z + w[:, 0:1] * g0_ref[...] + w[:, 1:2] * g1_ref[...])


def _combine_call(x2d, ws1, ws3, ws2, g0, g1, wts):
    nblk = S // TB
    hz = pl.pallas_call(
        _shared1_body,
        grid=(nblk,),
        in_specs=[
            pl.BlockSpec((TB, DIM), lambda i: (i, 0)),
            pl.BlockSpec((INTER, DIM), lambda i: (0, 0)),
            pl.BlockSpec((INTER, DIM), lambda i: (0, 0)),
        ],
        out_specs=pl.BlockSpec((TB, INTER), lambda i: (i, 0)),
        out_shape=jax.ShapeDtypeStruct((S, INTER), jnp.float32),
    )(x2d, ws1, ws3)
    return pl.pallas_call(
        _combine_body,
        grid=(nblk,),
        in_specs=[
            pl.BlockSpec((TB, INTER), lambda i: (i, 0)),
            pl.BlockSpec((DIM, INTER), lambda i: (0, 0)),
            pl.BlockSpec((TB, DIM), lambda i: (i, 0)),
            pl.BlockSpec((TB, DIM), lambda i: (i, 0)),
            pl.BlockSpec((TB, TOP_K), lambda i: (i, 0)),
        ],
        out_specs=pl.BlockSpec((TB, DIM), lambda i: (i, 0)),
        out_shape=jax.ShapeDtypeStruct((S, DIM), jnp.float32),
    )(hz, ws2, g0, g1, wts)


def kernel(x, gate_w, w1, w2, w3, ws1, ws2, ws3):
    shape = x.shape
    x2d = x.reshape(S, DIM)
    wts, pos, be = _gate_call(x2d, gate_w)
    pos0 = pos[:, 0]
    pos1 = pos[:, 1]
    xs = _make_sc_scatter()(x2d, pos0, pos1)
    ys = _ffn_call(be, xs, w1, w3, w2)
    g0, g1 = _make_sc_gather()(ys, pos0, pos1)
    y = _combine_call(x2d, ws1, ws3, ws2, g0, g1, wts)
    return y.reshape(shape)


# speedup vs baseline: 1.3671x; 1.0788x over previous
import functools

import jax
import jax.numpy as jnp
from jax import lax
from jax.experimental import pallas as pl
from jax.experimental.pallas import tpu as pltpu
from jax.experimental.pallas import tpu_sc as plsc

DIM = 2048
INTER = 1408
N_EXP = 16
TOP_K = 2
S = 2048
M = 512
NB = 24
CAP = M * NB
TB = 256


def _gate_body(x_ref, gw_ref, wts_ref, pos_ref, be_ref):
    xb = x_ref[...]
    gw = gw_ref[...]
    logits = lax.dot_general(xb, gw, (((1,), (1,)), ((), ())),
                             preferred_element_type=jnp.float32)
    m = jnp.max(logits, axis=-1, keepdims=True)
    ex = jnp.exp(logits - m)
    sc = ex / jnp.sum(ex, axis=-1, keepdims=True)

    lane = lax.broadcasted_iota(jnp.int32, (S, N_EXP), 1)
    v1 = jnp.max(sc, axis=-1, keepdims=True)
    i1 = jnp.min(jnp.where(sc == v1, lane, N_EXP), axis=-1, keepdims=True)
    sc2 = jnp.where(lane == i1, -1.0, sc)
    v2 = jnp.max(sc2, axis=-1, keepdims=True)
    i2 = jnp.min(jnp.where(sc2 == v2, lane, N_EXP), axis=-1, keepdims=True)
    wts_ref[...] = jnp.concatenate([v1, v2], axis=1)

    oh1 = lane == i1
    oh2 = lane == i2
    ohf = (oh1 | oh2).astype(jnp.float32)

    r = lax.broadcasted_iota(jnp.int32, (S, S), 0)
    c = lax.broadcasted_iota(jnp.int32, (S, S), 1)
    tri = (c < r).astype(jnp.float32)
    excl = lax.dot_general(tri, ohf, (((1,), (0,)), ((), ())),
                           preferred_element_type=jnp.float32)

    counts = jnp.sum(ohf, axis=0, keepdims=True)
    cp = jnp.floor((counts + (M - 1)) * (1.0 / M)) * M
    er = lax.broadcasted_iota(jnp.int32, (N_EXP, N_EXP), 0)
    ec = lax.broadcasted_iota(jnp.int32, (N_EXP, N_EXP), 1)
    ut = (er < ec).astype(jnp.float32)
    start = lax.dot_general(cp, ut, (((1,), (0,)), ((), ())),
                            preferred_element_type=jnp.float32)

    dest = start + excl
    p1 = jnp.sum(jnp.where(oh1, dest, 0.0), axis=-1, keepdims=True)
    p2 = jnp.sum(jnp.where(oh2, dest, 0.0), axis=-1, keepdims=True)
    pos_ref[...] = jnp.concatenate([p1, p2], axis=1).astype(jnp.int32)

    rb = (lax.broadcasted_iota(jnp.int32, (1, NB), 1) * M).astype(jnp.float32)
    rbt = jnp.broadcast_to(rb.reshape(NB, 1), (NB, N_EXP))
    sb = jnp.broadcast_to(start, (NB, N_EXP))
    cb = jnp.broadcast_to(cp, (NB, N_EXP))
    act = (rbt >= sb) & (rbt < sb + cb)
    ev = lax.broadcasted_iota(jnp.int32, (NB, N_EXP), 1).astype(jnp.float32)
    be = jnp.sum(jnp.where(act, ev + 1.0, 0.0), axis=-1, keepdims=False) - 1.0
    be_ref[...] = be.astype(jnp.int32).reshape(1, NB)


def _gate_call(x2d, gate_w):
    return pl.pallas_call(
        _gate_body,
        out_shape=(
            jax.ShapeDtypeStruct((S, TOP_K), jnp.float32),
            jax.ShapeDtypeStruct((S, TOP_K), jnp.int32),
            jax.ShapeDtypeStruct((1, NB), jnp.int32),
        ),
    )(x2d, gate_w)


def _make_sc_scatter():
    info = plsc.get_sparse_core_info()
    nc, ns = info.num_cores, info.num_subcores
    nw = nc * ns
    tpw = S // nw
    chunks = tpw // 16
    mesh = plsc.VectorSubcoreMesh(core_axis_name="c", subcore_axis_name="s")

    @functools.partial(
        pl.kernel, mesh=mesh,
        out_type=jax.ShapeDtypeStruct((CAP, DIM), jnp.float32),
        scratch_types=(
            [pltpu.VMEM((16, DIM), jnp.float32)] * 2
            + [pltpu.VMEM((16,), jnp.int32)] * (2 * chunks)
            + [pltpu.SemaphoreType.DMA] * 3
        ),
    )
    def scatter_kernel(x_hbm, p0_hbm, p1_hbm, xs_hbm, *scr):
        xa, xb = scr[0], scr[1]
        idx = scr[2:2 + 2 * chunks]
        lsem, ssem, isem = scr[-3], scr[-2], scr[-1]
        wid = lax.axis_index("s") * nc + lax.axis_index("c")
        base = wid * tpw
        for cc in range(chunks):
            pltpu.async_copy(p0_hbm.at[pl.ds(base + cc * 16, 16)],
                             idx[2 * cc], isem)
            pltpu.async_copy(p1_hbm.at[pl.ds(base + cc * 16, 16)],
                             idx[2 * cc + 1], isem)
        bufs = [xa, xb]
        lds = [None, None]
        lds[0] = pltpu.async_copy(x_hbm.at[pl.ds(base, 16)], xa, lsem)
        for cc in range(chunks):
            pltpu.make_async_copy(p0_hbm.at[pl.ds(base + cc * 16, 16)],
                                  idx[2 * cc], isem).wait()
            pltpu.make_async_copy(p1_hbm.at[pl.ds(base + cc * 16, 16)],
                                  idx[2 * cc + 1], isem).wait()
        sc_on_buf = [[], []]
        for cc in range(chunks):
            b = cc % 2
            nb_ = (cc + 1) % 2
            for s in sc_on_buf[nb_]:
                s.wait()
            sc_on_buf[nb_] = []
            if cc + 1 < chunks:
                lds[nb_] = pltpu.async_copy(
                    x_hbm.at[pl.ds(base + (cc + 1) * 16, 16)], bufs[nb_],
                    lsem)
            lds[b].wait()
            sc_on_buf[b] = [
                pltpu.async_copy(bufs[b], xs_hbm.at[idx[2 * cc]], ssem),
                pltpu.async_copy(bufs[b], xs_hbm.at[idx[2 * cc + 1]], ssem),
            ]
        for sl in sc_on_buf:
            for s in sl:
                s.wait()

    return scatter_kernel


def _ffn1_body(be_ref, xs_ref, w1_ref, w3_ref, h_ref):
    i = pl.program_id(0)

    @pl.when(be_ref[0, i] >= 0)
    def _():
        xb = xs_ref[...]
        h1 = lax.dot_general(xb, w1_ref[0], (((1,), (1,)), ((), ())),
                             preferred_element_type=jnp.float32)
        h3 = lax.dot_general(xb, w3_ref[0], (((1,), (1,)), ((), ())),
                             preferred_element_type=jnp.float32)
        h_ref[...] = ((h1 / (1.0 + jnp.exp(-h1))) * h3).astype(jnp.bfloat16)


def _ffn2_body(be_ref, h_ref, w2_ref, out_ref):
    i = pl.program_id(0)

    @pl.when(be_ref[0, i] >= 0)
    def _():
        hb = h_ref[...].astype(jnp.float32)
        out_ref[...] = lax.dot_general(
            hb, w2_ref[0], (((1,), (1,)), ((), ())),
            preferred_element_type=jnp.float32)


def _expert_spec():
    return lambda i, be: (jnp.maximum(be[0, i], 0), 0, 0)


def _row_in_spec(i, be):
    return (jnp.where(be[0, i] >= 0, i, 0), 0)


def _row_out_spec(i, be):
    return (jnp.where(be[0, i] >= 0, i, NB), 0)


def _ffn_call(be, xs, w1, w3, w2):
    gs1 = pltpu.PrefetchScalarGridSpec(
        num_scalar_prefetch=1,
        grid=(NB,),
        in_specs=[
            pl.BlockSpec((M, DIM), _row_in_spec),
            pl.BlockSpec((1, INTER, DIM), _expert_spec()),
            pl.BlockSpec((1, INTER, DIM), _expert_spec()),
        ],
        out_specs=pl.BlockSpec((M, INTER), _row_out_spec),
    )
    h = pl.pallas_call(
        _ffn1_body,
        grid_spec=gs1,
        out_shape=jax.ShapeDtypeStruct((CAP + M, INTER), jnp.bfloat16),
    )(be, xs, w1, w3)
    gs2 = pltpu.PrefetchScalarGridSpec(
        num_scalar_prefetch=1,
        grid=(NB,),
        in_specs=[
            pl.BlockSpec((M, INTER), _row_in_spec),
            pl.BlockSpec((1, DIM, INTER), _expert_spec()),
        ],
        out_specs=pl.BlockSpec((M, DIM), _row_out_spec),
    )
    return pl.pallas_call(
        _ffn2_body,
        grid_spec=gs2,
        out_shape=jax.ShapeDtypeStruct((CAP + M, DIM), jnp.float32),
    )(be, h, w2)


def _make_sc_gather():
    info = plsc.get_sparse_core_info()
    nc, ns = info.num_cores, info.num_subcores
    nw = nc * ns
    tpw = S // nw
    chunks = tpw // 16
    mesh = plsc.VectorSubcoreMesh(core_axis_name="c", subcore_axis_name="s")

    @functools.partial(
        pl.kernel, mesh=mesh,
        out_type=(
            jax.ShapeDtypeStruct((S, DIM), jnp.float32),
            jax.ShapeDtypeStruct((S, DIM), jnp.float32),
        ),
        scratch_types=(
            [pltpu.VMEM((16, DIM), jnp.float32)] * 2
            + [pltpu.VMEM((tpw,), jnp.int32)] * 2
            + [pltpu.SemaphoreType.DMA] * 3
        ),
    )
    def gather_kernel(ys_hbm, p0_hbm, p1_hbm, g0_hbm, g1_hbm, *scr):
        bufs = [scr[0], scr[1]]
        i0, i1 = scr[2], scr[3]
        gsem, ssem, isem = scr[-3], scr[-2], scr[-1]
        wid = lax.axis_index("s") * nc + lax.axis_index("c")
        base = wid * tpw
        pltpu.async_copy(p0_hbm.at[pl.ds(base, tpw)], i0, isem)
        pltpu.async_copy(p1_hbm.at[pl.ds(base, tpw)], i1, isem)
        pltpu.make_async_copy(p0_hbm.at[pl.ds(base, tpw)], i0, isem).wait()
        pltpu.make_async_copy(p1_hbm.at[pl.ds(base, tpw)], i1, isem).wait()
        stores = [[], []]
        for idx, g_hbm in ((i0, g0_hbm), (i1, g1_hbm)):
            prev = None
            for cc in range(chunks):
                b = cc % 2
                for st in stores[b]:
                    st.wait()
                stores[b] = []
                gc = pltpu.async_copy(
                    ys_hbm.at[idx.at[pl.ds(cc * 16, 16)]], bufs[b], gsem)
                if prev is not None:
                    pc, pg = prev
                    pg.wait()
                    stores[pc % 2] = [pltpu.async_copy(
                        bufs[pc % 2],
                        g_hbm.at[pl.ds(base + pc * 16, 16)], ssem)]
                prev = (cc, gc)
            pc, pg = prev
            pg.wait()
            stores[pc % 2] = [pltpu.async_copy(
                bufs[pc % 2], g_hbm.at[pl.ds(base + pc * 16, 16)], ssem)]
        for sl in stores:
            for st in sl:
                st.wait()

    return gather_kernel


def _shared1_body(x_ref, ws1_ref, ws3_ref, h_ref):
    xb = x_ref[...]
    h1 = lax.dot_general(xb, ws1_ref[...], (((1,), (1,)), ((), ())),
                         preferred_element_type=jnp.float32)
    h3 = lax.dot_general(xb, ws3_ref[...], (((1,), (1,)), ((), ())),
                         preferred_element_type=jnp.float32)
    h_ref[...] = ((h1 / (1.0 + jnp.exp(-h1))) * h3).astype(jnp.bfloat16)


def _combine_body(h_ref, ws2_ref, g0_ref, g1_ref, wts_ref, out_ref):
    z = lax.dot_general(h_ref[...].astype(jnp.float32), ws2_ref[...],
                        (((1,), (1,)), ((), ())),
                        preferred_element_type=jnp.float32)
    w = wts_ref[...]
    out_ref[...] = (z + w[:, 0:1] * g0_ref[...] + w[:, 1:2] * g1_ref[...])


def _combine_call(x2d, ws1, ws3, ws2, g0, g1, wts):
    nblk = S // TB
    hz = pl.pallas_call(
        _shared1_body,
        grid=(nblk,),
        in_specs=[
            pl.BlockSpec((TB, DIM), lambda i: (i, 0)),
            pl.BlockSpec((INTER, DIM), lambda i: (0, 0)),
            pl.BlockSpec((INTER, DIM), lambda i: (0, 0)),
        ],
        out_specs=pl.BlockSpec((TB, INTER), lambda i: (i, 0)),
        out_shape=jax.ShapeDtypeStruct((S, INTER), jnp.bfloat16),
    )(x2d, ws1, ws3)
    return pl.pallas_call(
        _combine_body,
        grid=(nblk,),
        in_specs=[
            pl.BlockSpec((TB, INTER), lambda i: (i, 0)),
            pl.BlockSpec((DIM, INTER), lambda i: (0, 0)),
            pl.BlockSpec((TB, DIM), lambda i: (i, 0)),
            pl.BlockSpec((TB, DIM), lambda i: (i, 0)),
            pl.BlockSpec((TB, TOP_K), lambda i: (i, 0)),
        ],
        out_specs=pl.BlockSpec((TB, DIM), lambda i: (i, 0)),
        out_shape=jax.ShapeDtypeStruct((S, DIM), jnp.float32),
    )(hz, ws2, g0, g1, wts)


def kernel(x, gate_w, w1, w2, w3, ws1, ws2, ws3):
    shape = x.shape
    x2d = x.reshape(S, DIM)
    wts, pos, be = _gate_call(x2d, gate_w)
    pos0 = pos[:, 0]
    pos1 = pos[:, 1]
    xs = _make_sc_scatter()(x2d, pos0, pos1)
    ys = _ffn_call(be, xs, w1, w3, w2)
    g0, g1 = _make_sc_gather()(ys, pos0, pos1)
    y = _combine_call(x2d, ws1, ws3, ws2, g0, g1, wts)
    return y.reshape(shape)
